# Initial kernel scaffold; baseline (speedup 1.0000x reference)
import jax
import jax.numpy as jnp
from jax.experimental import pallas as pl


def kernel(x, Wg, bg, W1, b1, W2, b2):
    raise NotImplementedError("write your pallas kernel here")


# speedup vs baseline: 2.4621x; 2.4621x over previous
import jax
import jax.numpy as jnp
from jax.experimental import pallas as pl
from jax.experimental.pallas import tpu as pltpu

E = 8
TOPK = 2
D = 1024
H = 2048
N = 2048
BH = 512
NH = H // BH


def _moe_dense_body(x_ref, wg_ref, bg_ref, w1_ref, b1_ref, w2_ref, b2_ref,
                    out_ref, scores_ref):
    e = pl.program_id(0)
    hb = pl.program_id(1)

    @pl.when((e == 0) & (hb == 0))
    def _init_gate():
        logits = jnp.dot(x_ref[...], wg_ref[...],
                         preferred_element_type=jnp.float32) + bg_ref[...]
        m = jnp.max(logits, axis=-1, keepdims=True)
        ex = jnp.exp(logits - m)
        scores_ref[...] = ex / jnp.sum(ex, axis=-1, keepdims=True)
        out_ref[...] = jnp.zeros_like(out_ref)

    s = scores_ref[...]
    ii = jax.lax.broadcasted_iota(jnp.int32, (N, E), 1)
    m1 = jnp.max(s, axis=-1, keepdims=True)
    idx1 = jnp.min(jnp.where(s == m1, ii, E), axis=-1, keepdims=True)
    s2 = jnp.where(ii == idx1, -jnp.inf, s)
    m2 = jnp.max(s2, axis=-1, keepdims=True)
    idx2 = jnp.min(jnp.where(s2 == m2, ii, E), axis=-1, keepdims=True)
    w_e = jnp.where(idx1 == e, m1, 0.0) + jnp.where(idx2 == e, m2, 0.0)

    h = jnp.dot(x_ref[...], w1_ref[0], preferred_element_type=jnp.float32)
    h = h + b1_ref[0]
    h = 0.5 * h * (1.0 + jax.lax.erf(h * 0.7071067811865476))
    acc = jnp.dot(h, w2_ref[0], preferred_element_type=jnp.float32)

    @pl.when(hb == 0)
    def _bias2():
        out_ref[...] += w_e * b2_ref[0]

    out_ref[...] += w_e * acc


@jax.jit
def kernel(x, Wg, bg, W1, b1, W2, b2):
    bg2 = bg.reshape(1, E)
    b1r = b1.reshape(E, 1, H)
    b2r = b2.reshape(E, 1, D)
    return pl.pallas_call(
        _moe_dense_body,
        grid=(E, NH),
        in_specs=[
            pl.BlockSpec((N, D), lambda e, hb: (0, 0)),
            pl.BlockSpec((D, E), lambda e, hb: (0, 0)),
            pl.BlockSpec((1, E), lambda e, hb: (0, 0)),
            pl.BlockSpec((1, D, BH), lambda e, hb: (e, 0, hb)),
            pl.BlockSpec((1, 1, BH), lambda e, hb: (e, 0, hb)),
            pl.BlockSpec((1, BH, D), lambda e, hb: (e, hb, 0)),
            pl.BlockSpec((1, 1, D), lambda e, hb: (e, 0, 0)),
        ],
        out_specs=pl.BlockSpec((N, D), lambda e, hb: (0, 0)),
        out_shape=jax.ShapeDtypeStruct((N, D), jnp.float32),
        scratch_shapes=[pltpu.VMEM((N, E), jnp.float32)],
        compiler_params=pltpu.CompilerParams(
            dimension_semantics=("arbitrary", "arbitrary"),
        ),
    )(x, Wg, bg2, W1, b1r, W2, b2r)


# speedup vs baseline: 3.1016x; 1.2597x over previous
import functools

import jax
import jax.numpy as jnp
from jax import lax
from jax.experimental import pallas as pl
from jax.experimental.pallas import tpu as pltpu
from jax.experimental.pallas import tpu_sc as plsc

E = 8
TOPK = 2
D = 1024
H = 2048
N = 2048
A = N * TOPK
BN = 128
NPAD = A + E * BN
NT = NPAD // BN
EPAD = 128
CB = 512

NC, NS = 2, 16
NW = NC * NS
APW = A // NW
TPW = N // NW
CH = 32


def _gate_body(x_ref, wg_ref, bg_ref, pos_ref, w0_ref, w1_ref, be_ref,
               oh_ref, cs_ref):
    logits = jnp.dot(x_ref[...], wg_ref[...],
                     preferred_element_type=jnp.float32) + bg_ref[...]
    m = jnp.max(logits, axis=-1, keepdims=True)
    ex = jnp.exp(logits - m)
    s = ex / jnp.sum(ex, axis=-1, keepdims=True)

    ii = lax.broadcasted_iota(jnp.int32, (N, E), 1)
    m1 = jnp.max(s, axis=-1, keepdims=True)
    idx1 = jnp.min(jnp.where(s == m1, ii, E), axis=-1, keepdims=True)
    s2 = jnp.where(ii == idx1, -jnp.inf, s)
    m2 = jnp.max(s2, axis=-1, keepdims=True)
    idx2 = jnp.min(jnp.where(s2 == m2, ii, E), axis=-1, keepdims=True)
    w0_ref[...] = m1
    w1_ref[...] = m2

    ef = jnp.concatenate([idx1, idx2], axis=0)
    ep = lax.broadcasted_iota(jnp.int32, (A, EPAD), 1)
    oh_ref[...] = (ep == ef).astype(jnp.float32)

    tri = (lax.broadcasted_iota(jnp.int32, (CB, CB), 1)
           <= lax.broadcasted_iota(jnp.int32, (CB, CB), 0)).astype(jnp.float32)
    run = jnp.zeros((1, EPAD), jnp.float32)
    for b in range(A // CB):
        blk = oh_ref[b * CB:(b + 1) * CB, :]
        loc = jnp.dot(tri, blk, preferred_element_type=jnp.float32) + run
        cs_ref[b * CB:(b + 1) * CB, :] = loc
        run = loc[CB - 1:CB, :]
    counts = run

    pc = jnp.floor((counts + (BN - 1)) * (1.0 / BN)) * BN
    su = (lax.broadcasted_iota(jnp.int32, (EPAD, EPAD), 0)
          < lax.broadcasted_iota(jnp.int32, (EPAD, EPAD), 1)).astype(jnp.float32)
    poff = jnp.dot(pc, su, preferred_element_type=jnp.float32)
    pend = poff + pc

    pos_f = jnp.sum(oh_ref[...] * (poff + cs_ref[...]), axis=-1,
                    keepdims=True) - 1.0
    pos_ref[...] = pos_f.astype(jnp.int32)

    tb = lax.broadcasted_iota(jnp.int32, (NT, EPAD), 0).astype(jnp.float32) * float(BN)
    be = jnp.sum((tb >= pend).astype(jnp.float32), axis=-1, keepdims=True)
    be_ref[...] = jnp.minimum(be, float(E - 1)).astype(jnp.int32)


def _gate_call(x, Wg, bg2):
    return pl.pallas_call(
        _gate_body,
        grid=(1,),
        in_specs=[
            pl.BlockSpec((N, D), lambda i: (0, 0)),
            pl.BlockSpec((D, E), lambda i: (0, 0)),
            pl.BlockSpec((1, E), lambda i: (0, 0)),
        ],
        out_specs=[
            pl.BlockSpec((A, 1), lambda i: (0, 0)),
            pl.BlockSpec((N, 1), lambda i: (0, 0)),
            pl.BlockSpec((N, 1), lambda i: (0, 0)),
            pl.BlockSpec((NT, 1), lambda i: (0, 0)),
        ],
        out_shape=[
            jax.ShapeDtypeStruct((A, 1), jnp.int32),
            jax.ShapeDtypeStruct((N, 1), jnp.float32),
            jax.ShapeDtypeStruct((N, 1), jnp.float32),
            jax.ShapeDtypeStruct((NT, 1), jnp.int32),
        ],
        scratch_shapes=[
            pltpu.VMEM((A, EPAD), jnp.float32),
            pltpu.VMEM((A, EPAD), jnp.float32),
        ],
        compiler_params=pltpu.CompilerParams(
            dimension_semantics=("arbitrary",),
        ),
    )(x, Wg, bg2)


def _sc_worker_id():
    return lax.axis_index("s") * NC + lax.axis_index("c")


def _sc_scatter_body(pos_hbm, x_hbm, xs_hbm, rows_v, idx_v, sem):
    base = _sc_worker_id() * APW
    for ch in range(APW // CH):
        a0 = base + ch * CH
        t0 = lax.rem(a0, N)
        pltpu.sync_copy(x_hbm.at[pl.ds(t0, CH)], rows_v)
        pltpu.sync_copy(pos_hbm.at[pl.ds(a0, CH)], idx_v)
        pltpu.async_copy(rows_v, xs_hbm.at[idx_v], sem).wait()


def _sc_scatter(pos_flat, x):
    f = pl.kernel(
        _sc_scatter_body,
        out_type=jax.ShapeDtypeStruct((NPAD, D), jnp.float32),
        mesh=plsc.VectorSubcoreMesh(core_axis_name="c", subcore_axis_name="s",
                                    num_cores=NC, num_subcores=NS),
        scratch_types=[
            pltpu.VMEM((CH, D), jnp.float32),
            pltpu.VMEM((CH,), jnp.int32),
            pltpu.SemaphoreType.DMA,
        ],
    )
    return f(pos_flat, x)


def _sc_gather_body(p0_hbm, p1_hbm, ys_hbm, g0_hbm, g1_hbm, rows_v, idx_v, sem):
    base = _sc_worker_id() * TPW
    for ch in range(TPW // CH):
        t0 = base + ch * CH
        pltpu.sync_copy(p0_hbm.at[pl.ds(t0, CH)], idx_v)
        pltpu.async_copy(ys_hbm.at[idx_v], rows_v, sem).wait()
        pltpu.sync_copy(rows_v, g0_hbm.at[pl.ds(t0, CH)])
        pltpu.sync_copy(p1_hbm.at[pl.ds(t0, CH)], idx_v)
        pltpu.async_copy(ys_hbm.at[idx_v], rows_v, sem).wait()
        pltpu.sync_copy(rows_v, g1_hbm.at[pl.ds(t0, CH)])


def _sc_gather(p0, p1, ys):
    f = pl.kernel(
        _sc_gather_body,
        out_type=(jax.ShapeDtypeStruct((N, D), jnp.float32),
                  jax.ShapeDtypeStruct((N, D), jnp.float32)),
        mesh=plsc.VectorSubcoreMesh(core_axis_name="c", subcore_axis_name="s",
                                    num_cores=NC, num_subcores=NS),
        scratch_types=[
            pltpu.VMEM((CH, D), jnp.float32),
            pltpu.VMEM((CH,), jnp.int32),
            pltpu.SemaphoreType.DMA,
        ],
    )
    return f(p0, p1, ys)


def _ffn_body(be_ref, xs_ref, w1_ref, b1_ref, w2_ref, b2_ref, ys_ref):
    h = jnp.dot(xs_ref[...], w1_ref[0], preferred_element_type=jnp.float32)
    h = h + b1_ref[0]
    h = 0.5 * h * (1.0 + lax.erf(h * 0.7071067811865476))
    ys_ref[...] = jnp.dot(h, w2_ref[0],
                          preferred_element_type=jnp.float32) + b2_ref[0]


def _ffn_call(be, xs, W1, b1r, W2, b2r):
    grid_spec = pltpu.PrefetchScalarGridSpec(
        num_scalar_prefetch=1,
        grid=(NT,),
        in_specs=[
            pl.BlockSpec((BN, D), lambda t, be: (t, 0)),
            pl.BlockSpec((1, D, H), lambda t, be: (be[t], 0, 0)),
            pl.BlockSpec((1, 1, H), lambda t, be: (be[t], 0, 0)),
            pl.BlockSpec((1, H, D), lambda t, be: (be[t], 0, 0)),
            pl.BlockSpec((1, 1, D), lambda t, be: (be[t], 0, 0)),
        ],
        out_specs=pl.BlockSpec((BN, D), lambda t, be: (t, 0)),
    )
    return pl.pallas_call(
        _ffn_body,
        grid_spec=grid_spec,
        out_shape=jax.ShapeDtypeStruct((NPAD, D), jnp.float32),
        compiler_params=pltpu.CompilerParams(
            dimension_semantics=("arbitrary",),
        ),
    )(be, xs, W1, b1r, W2, b2r)


def _combine_body(w0_ref, w1_ref, g0_ref, g1_ref, out_ref):
    out_ref[...] = w0_ref[...] * g0_ref[...] + w1_ref[...] * g1_ref[...]


def _combine_call(w0, w1, g0, g1):
    blk = 256
    return pl.pallas_call(
        _combine_body,
        grid=(N // blk,),
        in_specs=[
            pl.BlockSpec((blk, 1), lambda i: (i, 0)),
            pl.BlockSpec((blk, 1), lambda i: (i, 0)),
            pl.BlockSpec((blk, D), lambda i: (i, 0)),
            pl.BlockSpec((blk, D), lambda i: (i, 0)),
        ],
        out_specs=pl.BlockSpec((blk, D), lambda i: (i, 0)),
        out_shape=jax.ShapeDtypeStruct((N, D), jnp.float32),
    )(w0, w1, g0, g1)


@jax.jit
def kernel(x, Wg, bg, W1, b1, W2, b2):
    bg2 = bg.reshape(1, E)
    b1r = b1.reshape(E, 1, H)
    b2r = b2.reshape(E, 1, D)
    pos, w0, w1, be = _gate_call(x, Wg, bg2)
    pos_flat = pos.reshape(A)
    xs = _sc_scatter(pos_flat, x)
    ys = _ffn_call(be.reshape(NT), xs, W1, b1r, W2, b2r)
    g0, g1 = _sc_gather(pos_flat[:N], pos_flat[N:], ys)
    return _combine_call(w0, w1, g0, g1)
